# baseline (speedup 1.0000x reference)
import jax
import jax.numpy as jnp
from jax import lax
from jax.experimental import pallas as pl
from jax.experimental.pallas import tpu as pltpu
from jax.experimental.pallas import tpu_sc as plsc

_S = 64
_ROWS_PER_B = 32
_NGRP = 16
_NSC = 64
_NC, _NS, _L = 2, 16, 16
_NW = _NC * _NS
_BPW = _NSC // _NW
_RCH = 8


def _pool_mm_kernel(x_ref, w_ref, o_ref):
    g = o_ref.shape[0]
    blk = x_ref[...]
    blk = blk.reshape(g, _ROWS_PER_B, 2 * _NGRP, 128)
    s1 = jnp.sum(blk, axis=1)
    s2 = s1.reshape(g, _NGRP, 2, 128).sum(axis=2)
    sb = s2.astype(jnp.bfloat16)
    acc = jnp.zeros((g, 8), jnp.float32)
    for grp in range(_NGRP):
        acc = acc + jax.lax.dot_general(
            sb[:, grp, :], w_ref[grp],
            (((1,), (0,)), ((), ())),
            preferred_element_type=jnp.float32)
    o_ref[...] = acc


def _sc_kernel(x_hbm, w_hbm, o_hbm, buf0, buf1, acc, wv, pv, sem0, sem1):
    wid = lax.axis_index("s") * _NC + lax.axis_index("c")
    row_base = (512 - _NSC + wid * _BPW) * _ROWS_PER_B
    pltpu.sync_copy(w_hbm, wv)

    bufs = (buf0, buf1)
    sems = (sem0, sem1)
    nch = _ROWS_PER_B // _RCH
    total = _BPW * nch
    cps = [pltpu.async_copy(
        x_hbm.at[pl.ds(row_base + c * _RCH, _RCH)], bufs[c % 2], sems[c % 2])
        for c in range(min(2, total))]

    for c in range(total):
        bi, ci = divmod(c, nch)
        cps[c].wait()
        buf = bufs[c % 2]
        first = ci == 0

        @pl.loop(0, 2 * _NGRP)
        def _m(m):
            @pl.loop(0, 128, step=_L)
            def _l(l0):
                v = buf[0, m, pl.ds(l0, _L)]
                for r in range(1, _RCH):
                    v = v + buf[r, m, pl.ds(l0, _L)]
                if first:
                    acc[m, pl.ds(l0, _L)] = v
                else:
                    acc[m, pl.ds(l0, _L)] += v

        if ci == nch - 1:
            for j in range(5):
                pv[pl.ds(j * _L, _L)] = jnp.zeros((_L,), jnp.float32)

                @pl.loop(0, 2 * _NGRP)
                def _fm(m):
                    @pl.loop(0, 128, step=_L)
                    def _fl(l0):
                        pv[pl.ds(j * _L, _L)] += (
                            acc[m, pl.ds(l0, _L)] * wv[j, m, pl.ds(l0, _L)])
            for j in range(5, 8):
                pv[pl.ds(j * _L, _L)] = jnp.zeros((_L,), jnp.float32)
            pltpu.sync_copy(pv, o_hbm.at[wid * _BPW + bi])

        if c + 2 < total:
            cps.append(pltpu.async_copy(
                x_hbm.at[pl.ds(row_base + (c + 2) * _RCH, _RCH)],
                bufs[c % 2], sems[c % 2]))


def kernel(x, start_boundaries, W_loc, b_loc, W_conf, b_conf):
    B, C = x.shape[0], x.shape[1]
    ntc = B - _NSC
    xt = (x.transpose(0, 2, 3, 4, 1)
          .reshape(B * _ROWS_PER_B, 2, _NGRP, 128)
          .transpose(0, 2, 1, 3)
          .reshape(B * _ROWS_PER_B, 2 * _NGRP, 128))

    Wcat = jnp.concatenate([W_loc, W_conf], axis=0)
    W3 = jnp.pad((Wcat / _S).T.reshape(_NGRP, 128, 5),
                 ((0, 0), (0, 0), (0, 3))).astype(jnp.bfloat16)
    Wsc = jnp.repeat((Wcat / _S).reshape(5, _NGRP, 1, 128), 2, axis=2)
    Wsc = Wsc.reshape(5, 2 * _NGRP, 128)

    g = 32
    rb = g * _ROWS_PER_B
    out_tc = pl.pallas_call(
        _pool_mm_kernel,
        grid=(ntc // g,),
        in_specs=[
            pl.BlockSpec((rb, 2 * _NGRP, 128), lambda i: (i, 0, 0)),
            pl.BlockSpec((_NGRP, 128, 8), lambda i: (0, 0, 0)),
        ],
        out_specs=pl.BlockSpec((g, 8), lambda i: (i, 0)),
        out_shape=jax.ShapeDtypeStruct((ntc, 8), jnp.float32),
        compiler_params=pltpu.CompilerParams(
            dimension_semantics=("parallel",)),
    )(xt, W3)

    sc_fn = pl.kernel(
        _sc_kernel,
        mesh=plsc.VectorSubcoreMesh(core_axis_name="c", subcore_axis_name="s"),
        out_type=jax.ShapeDtypeStruct((_NSC, 8 * _L), jnp.float32),
        scratch_types=[
            pltpu.VMEM((_RCH, 2 * _NGRP, 128), jnp.float32),
            pltpu.VMEM((_RCH, 2 * _NGRP, 128), jnp.float32),
            pltpu.VMEM((2 * _NGRP, 128), jnp.float32),
            pltpu.VMEM((5, 2 * _NGRP, 128), jnp.float32),
            pltpu.VMEM((8 * _L,), jnp.float32),
            pltpu.SemaphoreType.DMA,
            pltpu.SemaphoreType.DMA,
        ])
    out_sc_p = sc_fn(xt, Wsc)
    out_sc = out_sc_p.reshape(_NSC, 8, _L).sum(-1)

    out = jnp.concatenate([out_tc, out_sc], axis=0)
    loc = out[:, :2] + b_loc[None, :]
    conf = out[:, 2:5] + b_conf[None, :]
    return loc, conf


# speedup vs baseline: 1.2285x; 1.2285x over previous
import jax
import jax.numpy as jnp
from jax.experimental import pallas as pl
from jax.experimental.pallas import tpu as pltpu

_S = 64
_ROWS_PER_B = 32
_NGRP = 16


def _pool_mm_kernel(xa_ref, xb_ref, w_ref, o_ref):
    g = o_ref.shape[0]
    h = g // 2

    def pooled(blk):
        b4 = blk.reshape(h, _ROWS_PER_B, 2 * _NGRP, 128)
        s1 = jnp.sum(b4, axis=1)
        return s1.reshape(h, _NGRP, 2, 128).sum(axis=2)

    sa = pooled(xa_ref[...]).astype(jnp.bfloat16)
    sb = pooled(xb_ref[...]).astype(jnp.bfloat16)
    acca = jnp.zeros((h, 8), jnp.float32)
    accb = jnp.zeros((h, 8), jnp.float32)
    for grp in range(_NGRP):
        dn = (((1,), (0,)), ((), ()))
        acca = acca + jax.lax.dot_general(
            sa[:, grp, :], w_ref[grp], dn, preferred_element_type=jnp.float32)
        accb = accb + jax.lax.dot_general(
            sb[:, grp, :], w_ref[grp], dn, preferred_element_type=jnp.float32)
    o_ref[...] = jnp.concatenate([acca, accb], axis=0)


def kernel(x, start_boundaries, W_loc, b_loc, W_conf, b_conf):
    B, C = x.shape[0], x.shape[1]
    xt = (x.transpose(0, 2, 3, 4, 1)
          .reshape(B * _ROWS_PER_B, 2, _NGRP, 128)
          .transpose(0, 2, 1, 3)
          .reshape(B * _ROWS_PER_B, 2 * _NGRP, 128))

    Wcat = jnp.concatenate([W_loc, W_conf], axis=0)
    W3 = jnp.pad((Wcat / _S).T.reshape(_NGRP, 128, 5),
                 ((0, 0), (0, 0), (0, 3))).astype(jnp.bfloat16)

    g = 32
    hb = (g // 2) * _ROWS_PER_B
    out = pl.pallas_call(
        _pool_mm_kernel,
        grid=(B // g,),
        in_specs=[
            pl.BlockSpec((hb, 2 * _NGRP, 128), lambda i: (2 * i, 0, 0)),
            pl.BlockSpec((hb, 2 * _NGRP, 128), lambda i: (2 * i + 1, 0, 0)),
            pl.BlockSpec((_NGRP, 128, 8), lambda i: (0, 0, 0)),
        ],
        out_specs=pl.BlockSpec((g, 8), lambda i: (i, 0)),
        out_shape=jax.ShapeDtypeStruct((B, 8), jnp.float32),
        compiler_params=pltpu.CompilerParams(
            dimension_semantics=("parallel",)),
    )(xt, xt, W3)

    loc = out[:, :2] + b_loc[None, :]
    conf = out[:, 2:5] + b_conf[None, :]
    return loc, conf


# speedup vs baseline: 1.2443x; 1.0129x over previous
import functools

import jax
import jax.numpy as jnp
from jax.experimental import pallas as pl
from jax.experimental.pallas import tpu as pltpu

_S = 64
_ROWS_PER_B = 32
_NGRP = 16
_NBLK = 16


def _pool_mm_kernel(x_ref, w_ref, b_ref, o_ref, *, g):
    i = pl.program_id(0)
    blk = x_ref[...]
    blk = blk.reshape(g, _ROWS_PER_B, 2 * _NGRP, 128)
    s1 = jnp.sum(blk, axis=1)
    s2 = s1.reshape(g, _NGRP, 2, 128).sum(axis=2)
    sb = s2.astype(jnp.bfloat16)
    acc = jnp.broadcast_to(b_ref[...], (8, g))
    for grp in range(_NGRP):
        acc = acc + jax.lax.dot_general(
            w_ref[:, grp * 128:(grp + 1) * 128], sb[:, grp, :],
            (((1,), (1,)), ((), ())),
            preferred_element_type=jnp.float32)
    for k in range(_NBLK):
        @pl.when(i == k)
        def _store():
            o_ref[:, k * g:(k + 1) * g] = acc


def kernel(x, start_boundaries, W_loc, b_loc, W_conf, b_conf):
    B, C = x.shape[0], x.shape[1]
    xt = (x.transpose(0, 2, 3, 4, 1)
          .reshape(B * _ROWS_PER_B, 2, _NGRP, 128)
          .transpose(0, 2, 1, 3)
          .reshape(B * _ROWS_PER_B, 2 * _NGRP, 128))

    zeros3 = jnp.zeros((3, C), jnp.float32)
    W8 = (jnp.concatenate([W_loc, W_conf, zeros3], axis=0) / _S
          ).astype(jnp.bfloat16)
    b8 = jnp.concatenate([b_loc, b_conf, jnp.zeros((3,), jnp.float32)]
                         ).reshape(8, 1)

    g = 32
    rb = g * _ROWS_PER_B
    out8 = pl.pallas_call(
        functools.partial(_pool_mm_kernel, g=g),
        grid=(B // g,),
        in_specs=[
            pl.BlockSpec((rb, 2 * _NGRP, 128), lambda i: (i, 0, 0)),
            pl.BlockSpec((8, C), lambda i: (0, 0)),
            pl.BlockSpec((8, 1), lambda i: (0, 0)),
        ],
        out_specs=pl.BlockSpec((8, B), lambda i: (0, 0)),
        out_shape=jax.ShapeDtypeStruct((8, B), jnp.float32),
        compiler_params=pltpu.CompilerParams(
            dimension_semantics=("arbitrary",)),
    )(xt, W8, b8)

    loc = out8[:2].T
    conf = out8[2:5].T
    return loc, conf


# speedup vs baseline: 1.2885x; 1.0355x over previous
import functools

import jax
import jax.numpy as jnp
from jax.experimental import pallas as pl
from jax.experimental.pallas import tpu as pltpu

_S = 64
_ROWS_PER_B = 32
_NGRP = 16
_NBLK = 16


def _pool_mm_kernel(x_ref, wl_ref, wc_ref, bl_ref, bc_ref,
                    ol_ref, oc_ref, *, g):
    i = pl.program_id(0)
    blk = x_ref[...]
    blk = blk.reshape(g, _ROWS_PER_B, 2 * _NGRP, 128)
    s1 = jnp.sum(blk, axis=1)
    s2 = s1.reshape(g, _NGRP, 2, 128).sum(axis=2)
    sb = (s2 * (1.0 / _S)).astype(jnp.bfloat16)
    wl = wl_ref[...].astype(jnp.bfloat16)
    wc = wc_ref[...].astype(jnp.bfloat16)
    accl = jnp.broadcast_to(bl_ref[...], (2, g))
    accc = jnp.broadcast_to(bc_ref[...], (3, g))
    dn = (((1,), (1,)), ((), ()))
    for grp in range(_NGRP):
        sl = slice(grp * 128, (grp + 1) * 128)
        accl = accl + jax.lax.dot_general(
            wl[:, sl], sb[:, grp, :], dn,
            preferred_element_type=jnp.float32)
        accc = accc + jax.lax.dot_general(
            wc[:, sl], sb[:, grp, :], dn,
            preferred_element_type=jnp.float32)
    for k in range(_NBLK):
        @pl.when(i == k)
        def _store():
            ol_ref[:, k * g:(k + 1) * g] = accl
            oc_ref[:, k * g:(k + 1) * g] = accc


def kernel(x, start_boundaries, W_loc, b_loc, W_conf, b_conf):
    B, C = x.shape[0], x.shape[1]
    xt = (x.transpose(0, 2, 3, 4, 1)
          .reshape(B * _ROWS_PER_B, 2, _NGRP, 128)
          .transpose(0, 2, 1, 3)
          .reshape(B * _ROWS_PER_B, 2 * _NGRP, 128))

    g = B // _NBLK
    rb = g * _ROWS_PER_B
    out2, out3 = pl.pallas_call(
        functools.partial(_pool_mm_kernel, g=g),
        grid=(_NBLK,),
        in_specs=[
            pl.BlockSpec((rb, 2 * _NGRP, 128), lambda i: (i, 0, 0)),
            pl.BlockSpec((2, C), lambda i: (0, 0)),
            pl.BlockSpec((3, C), lambda i: (0, 0)),
            pl.BlockSpec((2, 1), lambda i: (0, 0)),
            pl.BlockSpec((3, 1), lambda i: (0, 0)),
        ],
        out_specs=[
            pl.BlockSpec((2, B), lambda i: (0, 0)),
            pl.BlockSpec((3, B), lambda i: (0, 0)),
        ],
        out_shape=[
            jax.ShapeDtypeStruct((2, B), jnp.float32),
            jax.ShapeDtypeStruct((3, B), jnp.float32),
        ],
        compiler_params=pltpu.CompilerParams(
            dimension_semantics=("arbitrary",)),
    )(xt, W_loc, W_conf, b_loc.reshape(2, 1), b_conf.reshape(3, 1))

    return out2.T, out3.T
